# baseline (speedup 1.0000x reference)
import functools

import jax
import jax.numpy as jnp
from jax import lax
from jax.experimental import pallas as pl
from jax.experimental.pallas import tpu as pltpu
from jax.experimental.pallas import tpu_sc as plsc

D = 128
WP = 8
WT = 136
CH = 80
NSUB = 16


def _sc_segment_sums(T2, src3, dst3, nc, niter, n_out, acc_rows):
    rps = acc_rows // NSUB
    rpo = n_out // NSUB
    mesh = plsc.VectorSubcoreMesh(core_axis_name="c", subcore_axis_name="s")

    @functools.partial(
        pl.kernel,
        out_type=(
            jax.ShapeDtypeStruct((nc, n_out, D), jnp.float32),
            jax.ShapeDtypeStruct((nc, n_out, D), jnp.float32),
        ),
        mesh=mesh,
        compiler_params=pltpu.CompilerParams(use_tc_tiling_on_sc=False),
        scratch_types=[
            pltpu.VMEM((niter, CH), jnp.int32),
            pltpu.VMEM((niter, CH), jnp.int32),
            pltpu.VMEM((CH, WT), jnp.float32),
            pltpu.VMEM((CH, WT), jnp.float32),
            pltpu.VMEM_SHARED((acc_rows, WT), jnp.float32),
            pltpu.SemaphoreType.DMA,
            pltpu.SemaphoreType.DMA,
            pltpu.SemaphoreType.DMA,
        ],
    )
    def sc_kernel(t2_hbm, src_hbm, dst_hbm, outx_hbm, outp_hbm,
                  src_v, dst_v, rxa, rxb, acc,
                  semxa, semxb, sems):
        c = lax.axis_index("c")
        s = lax.axis_index("s")
        wid = s * nc + c

        zero16 = jnp.zeros((16,), jnp.float32)

        nzw = WT // 16 + 1

        def zbody(i, carry):
            r = i // nzw
            col = jnp.minimum((i % nzw) * 16, WT - 16)
            rxa[r, pl.ds(col, 16)] = zero16
            return carry

        lax.fori_loop(0, CH * nzw, zbody, 0)

        base = s * rps
        done = 0
        while done < rps:
            step = min(CH, rps - done)
            pltpu.sync_copy(rxa.at[pl.ds(0, step)],
                            acc.at[pl.ds(base + done, step)])
            done += step
        plsc.subcore_barrier()

        pltpu.sync_copy(src_hbm.at[wid], src_v)
        pltpu.sync_copy(dst_hbm.at[wid], dst_v)

        def gather(j, rx, semx):
            pltpu.async_copy(t2_hbm.at[src_v.at[j]], rx, semx)

        def drain_scatter(j, rx, semx):
            pltpu.make_async_copy(t2_hbm.at[src_v.at[j]], rx, semx).wait()
            pltpu.async_copy(rx, acc.at[dst_v.at[j]], sems, add=True).wait()

        gather(0, rxa, semxa)
        if niter > 1:
            gather(1, rxb, semxb)

        def body(k, carry):
            j = 2 * k
            drain_scatter(j, rxa, semxa)

            @pl.when(j + 2 < niter)
            def _():
                gather(j + 2, rxa, semxa)

            drain_scatter(j + 1, rxb, semxb)

            @pl.when(j + 3 < niter)
            def _():
                gather(j + 3, rxb, semxb)

            return carry

        lax.fori_loop(0, niter // 2, body, 0)
        if niter % 2:
            drain_scatter(niter - 1, rxa, semxa)
        plsc.subcore_barrier()

        ob = s * rpo
        pltpu.sync_copy(acc.at[pl.ds(ob, rpo), pl.ds(0, D)],
                        outx_hbm.at[c, pl.ds(ob, rpo)])
        pltpu.sync_copy(acc.at[pl.ds(ob, rpo), pl.ds(D, WP)],
                        outp_hbm.at[c, pl.ds(ob, rpo), pl.ds(0, WP)])

    return sc_kernel(T2, src3, dst3)


def _posdelta(x, pos8, Wh1, bh1r, Wh2p, bh2r, bn):
    n = x.shape[0]
    nb = n // bn

    def body(x_ref, pos_ref, wh1, bh1, wh2, bh2, o_ref):
        h1 = jnp.maximum(
            jnp.dot(x_ref[...], wh1[...], preferred_element_type=jnp.float32)
            + bh1[...], 0.0)
        delta = jnp.tanh(
            jnp.dot(h1, wh2[...], preferred_element_type=jnp.float32)
            + bh2[...])
        o_ref[...] = pos_ref[...] - delta[:, :WP]

    return pl.pallas_call(
        body,
        grid=(nb,),
        in_specs=[
            pl.BlockSpec((bn, D), lambda i: (i, 0)),
            pl.BlockSpec((bn, WP), lambda i: (i, 0)),
            pl.BlockSpec((D, D), lambda i: (0, 0)),
            pl.BlockSpec((1, D), lambda i: (0, 0)),
            pl.BlockSpec((D, D), lambda i: (0, 0)),
            pl.BlockSpec((1, D), lambda i: (0, 0)),
        ],
        out_specs=pl.BlockSpec((bn, WP), lambda i: (i, 0)),
        out_shape=jax.ShapeDtypeStruct((n, WP), jnp.float32),
    )(x, pos8, Wh1, bh1r, Wh2p, bh2r)


def _combine(x, pd8, Px, Pp, Wg1p, Wg1x, bg1r, Wg2, bg2r, bn):
    n = x.shape[0]
    nb = n // bn

    def body(x_ref, pd_ref, px_ref, pp_ref,
             wg1p, wg1x, bg1, wg2, bg2, o_ref):
        xb = x_ref[...]
        pd = pd_ref[...]
        Sx = px_ref[0] + px_ref[1]
        Sp = pp_ref[0][:, :WP] + pp_ref[1][:, :WP]
        cnt = Sp[:, 3:4]
        inv = 1.0 / jnp.maximum(cnt, 1.0)
        aggr_x = Sx * inv
        aggr_p = (Sp - cnt * pd) * inv
        g = jnp.maximum(
            jnp.dot(aggr_p, wg1p[...], preferred_element_type=jnp.float32)
            + jnp.dot(aggr_x, wg1x[...], preferred_element_type=jnp.float32)
            + bg1[...], 0.0)
        out = jnp.maximum(
            jnp.dot(g, wg2[...], preferred_element_type=jnp.float32)
            + bg2[...], 0.0)
        o_ref[...] = xb + out

    return pl.pallas_call(
        body,
        grid=(nb,),
        in_specs=[
            pl.BlockSpec((bn, D), lambda i: (i, 0)),
            pl.BlockSpec((bn, WP), lambda i: (i, 0)),
            pl.BlockSpec((2, bn, D), lambda i: (0, i, 0)),
            pl.BlockSpec((2, bn, D), lambda i: (0, i, 0)),
            pl.BlockSpec((WP, D), lambda i: (0, 0)),
            pl.BlockSpec((D, D), lambda i: (0, 0)),
            pl.BlockSpec((1, D), lambda i: (0, 0)),
            pl.BlockSpec((D, D), lambda i: (0, 0)),
            pl.BlockSpec((1, D), lambda i: (0, 0)),
        ],
        out_specs=pl.BlockSpec((bn, D), lambda i: (i, 0)),
        out_shape=jax.ShapeDtypeStruct((n, D), jnp.float32),
    )(x, pd8, Px, Pp, Wg1p, Wg1x, bg1r, Wg2, bg2r)


def _pick_bn(n):
    for b in range(min(512, n), 7, -8):
        if n % b == 0:
            return b
    return None


def kernel(x, pos, edge_index, Wh1, bh1, Wh2, bh2, Wg1, bg1, Wg2, bg2):
    N = x.shape[0]
    E = edge_index.shape[1]
    nc = plsc.get_sparse_core_info().num_cores
    nw = nc * NSUB

    bn = _pick_bn(N)
    if bn is None or N % NSUB != 0:
        npad = (-N) % (NSUB * 8)
        xw = jnp.pad(x, ((0, npad), (0, 0)))
        posw = jnp.pad(pos, ((0, npad), (0, 0)))
        nw_rows = N + npad
        bn = _pick_bn(nw_rows)
    else:
        xw, posw, nw_rows = x, pos, N

    acc_rows = (nw_rows + NSUB) // NSUB * NSUB

    T2 = jnp.concatenate(
        [xw, posw, jnp.ones((nw_rows, 1), jnp.float32),
         jnp.zeros((nw_rows, WT - D - 4), jnp.float32)], axis=1)
    pos8 = jnp.pad(posw, ((0, 0), (0, WP - 3)))

    ep = -(-E // (nw * CH)) * (nw * CH)
    src = edge_index[0]
    dst = edge_index[1]
    if ep != E:
        src = jnp.concatenate([src, jnp.zeros((ep - E,), jnp.int32)])
        dst = jnp.concatenate([dst, jnp.full((ep - E,), nw_rows, jnp.int32)])
    niter = ep // (nw * CH)
    src3 = src.reshape(nw, niter, CH)
    dst3 = dst.reshape(nw, niter, CH)

    bh1r = bh1.reshape(1, D)
    bh2r = jnp.pad(bh2, (0, D - 3)).reshape(1, D)
    bg1r = bg1.reshape(1, D)
    bg2r = bg2.reshape(1, D)
    Wh2p = jnp.pad(Wh2, ((0, 0), (0, D - 3)))
    Wg1p = jnp.pad(Wg1[:3], ((0, WP - 3), (0, 0)))
    Wg1x = Wg1[3:]

    pd8 = _posdelta(xw, pos8, Wh1, bh1r, Wh2p, bh2r, bn)
    Px, Pp = _sc_segment_sums(T2, src3, dst3, nc, niter, nw_rows, acc_rows)

    y = _combine(xw, pd8, Px, Pp, Wg1p, Wg1x, bg1r, Wg2, bg2r, bn)
    return y[:N] if nw_rows != N else y


# speedup vs baseline: 1.1262x; 1.1262x over previous
import functools

import jax
import jax.numpy as jnp
from jax import lax
from jax.experimental import pallas as pl
from jax.experimental.pallas import tpu as pltpu
from jax.experimental.pallas import tpu_sc as plsc

D = 128
WP = 8
CH = 80
NSUB = 16


def _sc_segment_sums(T, Tp, src3, dst3, nc, niter, n_out, acc_rows):
    rps = acc_rows // NSUB
    rpo = n_out // NSUB
    mesh = plsc.VectorSubcoreMesh(core_axis_name="c", subcore_axis_name="s")

    @functools.partial(
        pl.kernel,
        out_type=(
            jax.ShapeDtypeStruct((nc, n_out, D), jnp.float32),
            jax.ShapeDtypeStruct((nc, n_out, D), jnp.float32),
        ),
        mesh=mesh,
        compiler_params=pltpu.CompilerParams(use_tc_tiling_on_sc=False),
        scratch_types=[
            pltpu.VMEM((niter, CH), jnp.int32),
            pltpu.VMEM((niter, CH), jnp.int32),
            pltpu.VMEM((CH, D), jnp.float32),
            pltpu.VMEM((CH, D), jnp.float32),
            pltpu.VMEM((CH, WP), jnp.float32),
            pltpu.VMEM((CH, WP), jnp.float32),
            pltpu.VMEM_SHARED((acc_rows, D), jnp.float32),
            pltpu.VMEM_SHARED((acc_rows, WP), jnp.float32),
            pltpu.SemaphoreType.DMA,
            pltpu.SemaphoreType.DMA,
            pltpu.SemaphoreType.DMA,
            pltpu.SemaphoreType.DMA,
            pltpu.SemaphoreType.DMA,
            pltpu.SemaphoreType.DMA,
        ],
    )
    def sc_kernel(t_hbm, tp_hbm, src_hbm, dst_hbm, outx_hbm, outp_hbm,
                  src_v, dst_v, rxa, rxb, rpa, rpb, accx, accp,
                  semxa, semxb, sempa, sempb, semsx, semsp):
        c = lax.axis_index("c")
        s = lax.axis_index("s")
        wid = s * nc + c

        zero16 = jnp.zeros((16,), jnp.float32)

        def zbody(i, carry):
            r = i // (D // 16)
            col = (i % (D // 16)) * 16
            rxa[r, pl.ds(col, 16)] = zero16
            return carry

        lax.fori_loop(0, CH * (D // 16), zbody, 0)

        base = s * rps
        done = 0
        while done < rps:
            step = min(CH, rps - done)
            pltpu.sync_copy(rxa.at[pl.ds(0, step)],
                            accx.at[pl.ds(base + done, step)])
            pltpu.sync_copy(rxa.at[pl.ds(0, step), pl.ds(0, WP)],
                            accp.at[pl.ds(base + done, step)])
            done += step
        plsc.subcore_barrier()

        pltpu.sync_copy(src_hbm.at[wid], src_v)
        pltpu.sync_copy(dst_hbm.at[wid], dst_v)

        def gather(j, rx, rp, semx, semp):
            pltpu.async_copy(t_hbm.at[src_v.at[j]], rx, semx)
            pltpu.async_copy(tp_hbm.at[src_v.at[j]], rp, semp)

        def drain_scatter(j, rx, rp, semx, semp):
            pltpu.make_async_copy(t_hbm.at[src_v.at[j]], rx, semx).wait()
            pltpu.make_async_copy(tp_hbm.at[src_v.at[j]], rp, semp).wait()
            cx = pltpu.async_copy(rx, accx.at[dst_v.at[j]], semsx, add=True)
            cp = pltpu.async_copy(rp, accp.at[dst_v.at[j]], semsp, add=True)
            cx.wait()
            cp.wait()

        gather(0, rxa, rpa, semxa, sempa)
        if niter > 1:
            gather(1, rxb, rpb, semxb, sempb)

        def body(k, carry):
            j = 2 * k
            drain_scatter(j, rxa, rpa, semxa, sempa)

            @pl.when(j + 2 < niter)
            def _():
                gather(j + 2, rxa, rpa, semxa, sempa)

            drain_scatter(j + 1, rxb, rpb, semxb, sempb)

            @pl.when(j + 3 < niter)
            def _():
                gather(j + 3, rxb, rpb, semxb, sempb)

            return carry

        lax.fori_loop(0, niter // 2, body, 0)
        if niter % 2:
            drain_scatter(niter - 1, rxa, rpa, semxa, sempa)
        plsc.subcore_barrier()

        ob = s * rpo
        pltpu.sync_copy(accx.at[pl.ds(ob, rpo)],
                        outx_hbm.at[c, pl.ds(ob, rpo)])
        pltpu.sync_copy(accp.at[pl.ds(ob, rpo)],
                        outp_hbm.at[c, pl.ds(ob, rpo), pl.ds(0, WP)])

    return sc_kernel(T, Tp, src3, dst3)


def _posdelta(x, pos8, Wh1, bh1r, Wh2p, bh2r, bn):
    n = x.shape[0]
    nb = n // bn

    def body(x_ref, pos_ref, wh1, bh1, wh2, bh2, o_ref):
        h1 = jnp.maximum(
            jnp.dot(x_ref[...], wh1[...], preferred_element_type=jnp.float32)
            + bh1[...], 0.0)
        delta = jnp.tanh(
            jnp.dot(h1, wh2[...], preferred_element_type=jnp.float32)
            + bh2[...])
        o_ref[...] = pos_ref[...] - delta[:, :WP]

    return pl.pallas_call(
        body,
        grid=(nb,),
        in_specs=[
            pl.BlockSpec((bn, D), lambda i: (i, 0)),
            pl.BlockSpec((bn, WP), lambda i: (i, 0)),
            pl.BlockSpec((D, D), lambda i: (0, 0)),
            pl.BlockSpec((1, D), lambda i: (0, 0)),
            pl.BlockSpec((D, D), lambda i: (0, 0)),
            pl.BlockSpec((1, D), lambda i: (0, 0)),
        ],
        out_specs=pl.BlockSpec((bn, WP), lambda i: (i, 0)),
        out_shape=jax.ShapeDtypeStruct((n, WP), jnp.float32),
    )(x, pos8, Wh1, bh1r, Wh2p, bh2r)


def _combine(x, pd8, Px, Pp, Wg1p, Wg1x, bg1r, Wg2, bg2r, bn):
    n = x.shape[0]
    nb = n // bn

    def body(x_ref, pd_ref, px_ref, pp_ref,
             wg1p, wg1x, bg1, wg2, bg2, o_ref):
        xb = x_ref[...]
        pd = pd_ref[...]
        Sx = px_ref[0] + px_ref[1]
        Sp = pp_ref[0][:, :WP] + pp_ref[1][:, :WP]
        cnt = Sp[:, 3:4]
        inv = 1.0 / jnp.maximum(cnt, 1.0)
        aggr_x = Sx * inv
        aggr_p = (Sp - cnt * pd) * inv
        g = jnp.maximum(
            jnp.dot(aggr_p, wg1p[...], preferred_element_type=jnp.float32)
            + jnp.dot(aggr_x, wg1x[...], preferred_element_type=jnp.float32)
            + bg1[...], 0.0)
        out = jnp.maximum(
            jnp.dot(g, wg2[...], preferred_element_type=jnp.float32)
            + bg2[...], 0.0)
        o_ref[...] = xb + out

    return pl.pallas_call(
        body,
        grid=(nb,),
        in_specs=[
            pl.BlockSpec((bn, D), lambda i: (i, 0)),
            pl.BlockSpec((bn, WP), lambda i: (i, 0)),
            pl.BlockSpec((2, bn, D), lambda i: (0, i, 0)),
            pl.BlockSpec((2, bn, D), lambda i: (0, i, 0)),
            pl.BlockSpec((WP, D), lambda i: (0, 0)),
            pl.BlockSpec((D, D), lambda i: (0, 0)),
            pl.BlockSpec((1, D), lambda i: (0, 0)),
            pl.BlockSpec((D, D), lambda i: (0, 0)),
            pl.BlockSpec((1, D), lambda i: (0, 0)),
        ],
        out_specs=pl.BlockSpec((bn, D), lambda i: (i, 0)),
        out_shape=jax.ShapeDtypeStruct((n, D), jnp.float32),
    )(x, pd8, Px, Pp, Wg1p, Wg1x, bg1r, Wg2, bg2r)


def _pick_bn(n):
    for b in range(min(2000, n), 7, -8):
        if n % b == 0:
            return b
    return None


def kernel(x, pos, edge_index, Wh1, bh1, Wh2, bh2, Wg1, bg1, Wg2, bg2):
    N = x.shape[0]
    E = edge_index.shape[1]
    nc = plsc.get_sparse_core_info().num_cores
    nw = nc * NSUB

    bn = _pick_bn(N)
    if bn is None or N % NSUB != 0:
        npad = (-N) % (NSUB * 8)
        xw = jnp.pad(x, ((0, npad), (0, 0)))
        posw = jnp.pad(pos, ((0, npad), (0, 0)))
        nw_rows = N + npad
        bn = _pick_bn(nw_rows)
    else:
        xw, posw, nw_rows = x, pos, N

    acc_rows = (nw_rows + NSUB) // NSUB * NSUB

    Tp = jnp.concatenate([posw, jnp.ones((nw_rows, 1), jnp.float32)], axis=1)
    Tp = jnp.pad(Tp, ((0, 0), (0, WP - 4)))
    pos8 = jnp.pad(posw, ((0, 0), (0, WP - 3)))

    ep = -(-E // (nw * CH)) * (nw * CH)
    src = edge_index[0]
    dst = edge_index[1]
    if ep != E:
        src = jnp.concatenate([src, jnp.zeros((ep - E,), jnp.int32)])
        dst = jnp.concatenate([dst, jnp.full((ep - E,), nw_rows, jnp.int32)])
    niter = ep // (nw * CH)
    src3 = src.reshape(nw, niter, CH)
    dst3 = dst.reshape(nw, niter, CH)

    bh1r = bh1.reshape(1, D)
    bh2r = jnp.pad(bh2, (0, D - 3)).reshape(1, D)
    bg1r = bg1.reshape(1, D)
    bg2r = bg2.reshape(1, D)
    Wh2p = jnp.pad(Wh2, ((0, 0), (0, D - 3)))
    Wg1p = jnp.pad(Wg1[:3], ((0, WP - 3), (0, 0)))
    Wg1x = Wg1[3:]

    pd8 = _posdelta(xw, pos8, Wh1, bh1r, Wh2p, bh2r, bn)
    Px, Pp = _sc_segment_sums(xw, Tp, src3, dst3, nc, niter, nw_rows,
                              acc_rows)

    y = _combine(xw, pd8, Px, Pp, Wg1p, Wg1x, bg1r, Wg2, bg2r, bn)
    return y[:N] if nw_rows != N else y
